# baseline (speedup 1.0000x reference)
import functools

import jax
import jax.numpy as jnp
from jax.experimental import pallas as pl
from jax.experimental.pallas import tpu as pltpu
from jax.experimental.pallas import tpu_sc as plsc

NUM_EXPERTS = 8
TOP_K = 2
HIDDEN = 1024
INTER = 4096
TOKENS = 2048

BM = 512
BI = 1024
NT = (TOKENS * TOP_K) // BM + NUM_EXPERTS
P = NT * BM
NI = INTER // BI


def _lane_cumsum(a):
    n = a.shape[-1]
    k = 1
    while k < n:
        shifted = jnp.concatenate(
            [jnp.zeros(a.shape[:-1] + (k,), a.dtype), a[..., :-k]], axis=-1)
        a = a + shifted
        k *= 2
    return a


def _router_kernel(x_ref, gw_ref, dst_ref, wts_ref, aux_ref):
    logits = jax.lax.dot_general(
        gw_ref[...], x_ref[...], (((1,), (1,)), ((), ())),
        preferred_element_type=jnp.float32)
    iota_e = jax.lax.broadcasted_iota(jnp.int32, logits.shape, 0)
    big = jnp.float32(1e30)
    l0 = jnp.max(logits, axis=0, keepdims=True)
    a0 = jnp.min(jnp.where(logits == l0, iota_e, NUM_EXPERTS), axis=0,
                 keepdims=True)
    masked = jnp.where(iota_e == a0, -big, logits)
    l1 = jnp.max(masked, axis=0, keepdims=True)
    a1 = jnp.min(jnp.where(masked == l1, iota_e, NUM_EXPERTS), axis=0,
                 keepdims=True)
    w0 = jax.nn.sigmoid(l0 - l1)
    wts_ref[...] = jnp.where(iota_e == 0, w0,
                             jnp.where(iota_e == 1, 1.0 - w0, 0.0))

    e_all = jnp.concatenate([a0, a1], axis=1)
    iota_e2 = jax.lax.broadcasted_iota(jnp.int32, (NUM_EXPERTS, 2 * TOKENS), 0)
    oh = (iota_e2 == e_all).astype(jnp.float32)
    inc = _lane_cumsum(oh)
    rank = inc - oh
    counts = inc[:, -1:]
    padded = jnp.ceil(counts / BM) * BM
    iota_r = jax.lax.broadcasted_iota(
        jnp.int32, (NUM_EXPERTS, NUM_EXPERTS), 0)
    iota_c = jax.lax.broadcasted_iota(
        jnp.int32, (NUM_EXPERTS, NUM_EXPERTS), 1)
    l_strict = (iota_c < iota_r).astype(jnp.float32)
    pad_start = jnp.dot(l_strict, padded,
                        preferred_element_type=jnp.float32)
    dst = jnp.sum(oh * (rank + pad_start), axis=0, keepdims=True)
    dst_ref[...] = jnp.broadcast_to(dst, (NUM_EXPERTS, 2 * TOKENS)).astype(
        jnp.int32)

    pos = jax.lax.broadcasted_iota(
        jnp.int32, (NUM_EXPERTS, 128), 1).astype(jnp.float32) * BM
    ep = jnp.sum((pos >= pad_start).astype(jnp.float32), axis=0,
                 keepdims=True) - 1.0
    total = jnp.sum(padded)
    valid = (pos[0:1, :] < total).astype(jnp.float32)
    iota_a = jax.lax.broadcasted_iota(jnp.int32, (NUM_EXPERTS, 128), 0)
    aux_ref[...] = jnp.where(iota_a == 0, ep,
                             jnp.where(iota_a == 1, valid, 0.0)).astype(
                                 jnp.int32)


def _router(hidden_states, gate_w):
    return pl.pallas_call(
        _router_kernel,
        out_shape=[
            jax.ShapeDtypeStruct((NUM_EXPERTS, 2 * TOKENS), jnp.int32),
            jax.ShapeDtypeStruct((NUM_EXPERTS, TOKENS), jnp.float32),
            jax.ShapeDtypeStruct((NUM_EXPERTS, 128), jnp.int32),
        ],
    )(hidden_states, gate_w)


ASSIGN = TOKENS * TOP_K
NW = 32
APW = ASSIGN // NW
CH = 64
NCH = APW // CH


def _sc_route(hidden_states, dst):
    mesh = plsc.VectorSubcoreMesh(core_axis_name="c", subcore_axis_name="s")

    @functools.partial(
        pl.kernel, mesh=mesh,
        out_type=jax.ShapeDtypeStruct((P, HIDDEN), jnp.float32),
        scratch_types=[
            pltpu.VMEM((CH,), jnp.int32),
            pltpu.VMEM((CH, HIDDEN), jnp.float32),
            pltpu.SemaphoreType.DMA,
        ],
    )
    def k(x_hbm, dst_hbm, xs_hbm, idx_v, rows_v, sem):
        wid = jax.lax.axis_index("s") * 2 + jax.lax.axis_index("c")
        base = wid * APW
        for c in range(NCH):
            off = base + c * CH
            pltpu.sync_copy(dst_hbm.at[pl.ds(off, CH)], idx_v)
            src = jax.lax.rem(off, TOKENS)
            pltpu.sync_copy(x_hbm.at[pl.ds(src, CH)], rows_v)
            pltpu.async_copy(rows_v, xs_hbm.at[idx_v], sem).wait()

    return k(hidden_states, dst)


def _mlp_kernel(expert_ref, valid_ref, xs_ref, w1_ref, w3_ref, w2_ref, out_ref,
                acc_ref):
    t = pl.program_id(0)
    i = pl.program_id(1)

    @pl.when(valid_ref[t] != 0)
    def _():
        x = xs_ref[...]
        dn = (((1,), (1,)), ((), ()))
        a = jax.lax.dot_general(x, w1_ref[0], dn,
                                preferred_element_type=jnp.float32)
        b = jax.lax.dot_general(x, w3_ref[0], dn,
                                preferred_element_type=jnp.float32)
        h = jax.nn.silu(a) * b
        prod = jax.lax.dot_general(h, w2_ref[0], dn,
                                   preferred_element_type=jnp.float32)

        @pl.when(i == 0)
        def _():
            acc_ref[...] = prod

        @pl.when(i != 0)
        def _():
            acc_ref[...] += prod

        @pl.when(i == NI - 1)
        def _():
            out_ref[...] = acc_ref[...]


def _grouped_mlp(xs, tile_expert, tile_valid, w1, w3, w2):
    grid_spec = pltpu.PrefetchScalarGridSpec(
        num_scalar_prefetch=2,
        grid=(NT, NI),
        in_specs=[
            pl.BlockSpec((BM, HIDDEN), lambda t, i, e, v: (t, 0)),
            pl.BlockSpec((1, BI, HIDDEN), lambda t, i, e, v: (e[t], i, 0)),
            pl.BlockSpec((1, BI, HIDDEN), lambda t, i, e, v: (e[t], i, 0)),
            pl.BlockSpec((1, HIDDEN, BI), lambda t, i, e, v: (e[t], 0, i)),
        ],
        out_specs=pl.BlockSpec((BM, HIDDEN), lambda t, i, e, v: (t, 0)),
        scratch_shapes=[pltpu.VMEM((BM, HIDDEN), jnp.float32)],
    )
    return pl.pallas_call(
        _mlp_kernel,
        grid_spec=grid_spec,
        out_shape=jax.ShapeDtypeStruct((P, HIDDEN), jnp.float32),
        compiler_params=pltpu.CompilerParams(
            dimension_semantics=("arbitrary", "arbitrary"),
        ),
    )(tile_expert, tile_valid, xs, w1, w3, w2)


def kernel(hidden_states, gate_w, w1, w3, w2):
    dst8, wtsT, aux = _router(hidden_states, gate_w)

    dst = dst8[0]

    xs = _sc_route(hidden_states, dst)

    tile_expert = aux[0, :NT]
    tile_valid = aux[1, :NT]

    y = _grouped_mlp(xs, tile_expert, tile_valid, w1, w3, w2)

    out = (wtsT[0][:, None] * y[dst[:TOKENS]]
           + wtsT[1][:, None] * y[dst[TOKENS:]])
    return out


# speedup vs baseline: 1.0069x; 1.0069x over previous
import functools

import jax
import jax.numpy as jnp
from jax.experimental import pallas as pl
from jax.experimental.pallas import tpu as pltpu
from jax.experimental.pallas import tpu_sc as plsc

NUM_EXPERTS = 8
TOP_K = 2
HIDDEN = 1024
INTER = 4096
TOKENS = 2048

BM = 512
BI = 1024
NT = (TOKENS * TOP_K) // BM + NUM_EXPERTS
P = NT * BM
NI = INTER // BI


def _lane_cumsum(a):
    n = a.shape[-1]
    k = 1
    while k < n:
        shifted = jnp.concatenate(
            [jnp.zeros(a.shape[:-1] + (k,), a.dtype), a[..., :-k]], axis=-1)
        a = a + shifted
        k *= 2
    return a


def _router_kernel(x_ref, gw_ref, dst_ref, wts_ref, aux_ref):
    logits = jax.lax.dot_general(
        gw_ref[...], x_ref[...], (((1,), (1,)), ((), ())),
        preferred_element_type=jnp.float32)
    iota_e = jax.lax.broadcasted_iota(jnp.int32, logits.shape, 0)
    big = jnp.float32(1e30)
    l0 = jnp.max(logits, axis=0, keepdims=True)
    a0 = jnp.min(jnp.where(logits == l0, iota_e, NUM_EXPERTS), axis=0,
                 keepdims=True)
    masked = jnp.where(iota_e == a0, -big, logits)
    l1 = jnp.max(masked, axis=0, keepdims=True)
    a1 = jnp.min(jnp.where(masked == l1, iota_e, NUM_EXPERTS), axis=0,
                 keepdims=True)
    w0 = jax.nn.sigmoid(l0 - l1)
    wts_ref[...] = jnp.where(iota_e == 0, w0,
                             jnp.where(iota_e == 1, 1.0 - w0, 0.0))

    e_all = jnp.concatenate([a0, a1], axis=1)
    iota_e2 = jax.lax.broadcasted_iota(jnp.int32, (NUM_EXPERTS, 2 * TOKENS), 0)
    oh = (iota_e2 == e_all).astype(jnp.float32)
    inc = _lane_cumsum(oh)
    rank = inc - oh
    counts = inc[:, -1:]
    padded = jnp.ceil(counts / BM) * BM
    iota_r = jax.lax.broadcasted_iota(
        jnp.int32, (NUM_EXPERTS, NUM_EXPERTS), 0)
    iota_c = jax.lax.broadcasted_iota(
        jnp.int32, (NUM_EXPERTS, NUM_EXPERTS), 1)
    l_strict = (iota_c < iota_r).astype(jnp.float32)
    pad_start = jnp.dot(l_strict, padded,
                        preferred_element_type=jnp.float32)
    dst = jnp.sum(oh * (rank + pad_start), axis=0, keepdims=True)
    dst_ref[...] = jnp.broadcast_to(dst, (NUM_EXPERTS, 2 * TOKENS)).astype(
        jnp.int32)

    pos = jax.lax.broadcasted_iota(
        jnp.int32, (NUM_EXPERTS, 128), 1).astype(jnp.float32) * BM
    ep = jnp.sum((pos >= pad_start).astype(jnp.float32), axis=0,
                 keepdims=True) - 1.0
    total = jnp.sum(padded)
    valid = (pos[0:1, :] < total).astype(jnp.float32)
    iota_a = jax.lax.broadcasted_iota(jnp.int32, (NUM_EXPERTS, 128), 0)
    aux_ref[...] = jnp.where(iota_a == 0, ep,
                             jnp.where(iota_a == 1, valid, 0.0)).astype(
                                 jnp.int32)


def _router(hidden_states, gate_w):
    return pl.pallas_call(
        _router_kernel,
        out_shape=[
            jax.ShapeDtypeStruct((NUM_EXPERTS, 2 * TOKENS), jnp.int32),
            jax.ShapeDtypeStruct((NUM_EXPERTS, TOKENS), jnp.float32),
            jax.ShapeDtypeStruct((NUM_EXPERTS, 128), jnp.int32),
        ],
    )(hidden_states, gate_w)


ASSIGN = TOKENS * TOP_K
NW = 32
APW = ASSIGN // NW
CH = 64
NCH = APW // CH


def _sc_route(hidden_states, dst):
    mesh = plsc.VectorSubcoreMesh(core_axis_name="c", subcore_axis_name="s")

    @functools.partial(
        pl.kernel, mesh=mesh,
        out_type=jax.ShapeDtypeStruct((P, HIDDEN), jnp.float32),
        scratch_types=[
            pltpu.VMEM((CH,), jnp.int32),
            pltpu.VMEM((CH, HIDDEN), jnp.float32),
            pltpu.SemaphoreType.DMA,
        ],
    )
    def k(x_hbm, dst_hbm, xs_hbm, idx_v, rows_v, sem):
        wid = jax.lax.axis_index("s") * 2 + jax.lax.axis_index("c")
        base = wid * APW
        for c in range(NCH):
            off = base + c * CH
            pltpu.sync_copy(dst_hbm.at[pl.ds(off, CH)], idx_v)
            src = jax.lax.rem(off, TOKENS)
            pltpu.sync_copy(x_hbm.at[pl.ds(src, CH)], rows_v)
            pltpu.async_copy(rows_v, xs_hbm.at[idx_v], sem).wait()

    return k(hidden_states, dst)


def _mlp_kernel(expert_ref, valid_ref, xs_ref, w1_ref, w3_ref, w2_ref, out_ref,
                acc_ref):
    t = pl.program_id(0)
    i = pl.program_id(1)

    @pl.when(valid_ref[t] != 0)
    def _():
        @pl.when(i == 0)
        def _():
            acc_ref[...] = jnp.zeros_like(acc_ref)

        x = xs_ref[...]
        dn = (((1,), (1,)), ((), ()))
        a = jax.lax.dot_general(x, w1_ref[0], dn,
                                preferred_element_type=jnp.float32)
        b = jax.lax.dot_general(x, w3_ref[0], dn,
                                preferred_element_type=jnp.float32)
        h = jax.nn.silu(a) * b
        acc_ref[...] += jax.lax.dot_general(
            h, w2_ref[0], dn, preferred_element_type=jnp.float32)

        @pl.when(i == NI - 1)
        def _():
            out_ref[...] = acc_ref[...]


def _grouped_mlp(xs, tile_expert, tile_valid, w1, w3, w2):
    grid_spec = pltpu.PrefetchScalarGridSpec(
        num_scalar_prefetch=2,
        grid=(NT, NI),
        in_specs=[
            pl.BlockSpec((BM, HIDDEN), lambda t, i, e, v: (t, 0)),
            pl.BlockSpec((1, BI, HIDDEN), lambda t, i, e, v: (e[t], i, 0)),
            pl.BlockSpec((1, BI, HIDDEN), lambda t, i, e, v: (e[t], i, 0)),
            pl.BlockSpec((1, HIDDEN, BI), lambda t, i, e, v: (e[t], 0, i)),
        ],
        out_specs=pl.BlockSpec((BM, HIDDEN), lambda t, i, e, v: (t, 0)),
        scratch_shapes=[pltpu.VMEM((BM, HIDDEN), jnp.float32)],
    )
    return pl.pallas_call(
        _mlp_kernel,
        grid_spec=grid_spec,
        out_shape=jax.ShapeDtypeStruct((P, HIDDEN), jnp.float32),
        compiler_params=pltpu.CompilerParams(
            dimension_semantics=("arbitrary", "arbitrary"),
        ),
    )(tile_expert, tile_valid, xs, w1, w3, w2)


def kernel(hidden_states, gate_w, w1, w3, w2):
    dst8, wtsT, aux = _router(hidden_states, gate_w)

    dst = dst8[0]

    xs = _sc_route(hidden_states, dst)

    tile_expert = aux[0, :NT]
    tile_valid = aux[1, :NT]

    y = _grouped_mlp(xs, tile_expert, tile_valid, w1, w3, w2)

    out = (wtsT[0][:, None] * y[dst[:TOKENS]]
           + wtsT[1][:, None] * y[dst[TOKENS:]])
    return out
